# baseline (speedup 1.0000x reference)
import jax
import jax.numpy as jnp
from jax.experimental import pallas as pl

_NUM_CLASSES = 8192
_Z_DIM = 256
_BLOCK_ROWS = 1024


def _copy_body(a_ref, o_ref):
    o_ref[...] = a_ref[...]


def kernel(_, anchor):
    grid = (_NUM_CLASSES // _BLOCK_ROWS,)
    return pl.pallas_call(
        _copy_body,
        grid=grid,
        in_specs=[pl.BlockSpec((_BLOCK_ROWS, _Z_DIM), lambda i: (i, 0))],
        out_specs=pl.BlockSpec((_BLOCK_ROWS, _Z_DIM), lambda i: (i, 0)),
        out_shape=jax.ShapeDtypeStruct((_NUM_CLASSES, _Z_DIM), jnp.float32),
    )(anchor)
